# baseline (speedup 1.0000x reference)
import functools

import jax
import jax.numpy as jnp
from jax import lax
from jax.experimental import pallas as pl
from jax.experimental.pallas import tpu as pltpu
from jax.experimental.pallas import tpu_sc as plsc

NT = 16
CHUNK = 128
HALF = 128
IBLK = 40
SPLIT = 1


def _sc_mesh():
    return plsc.VectorSubcoreMesh(core_axis_name="c", subcore_axis_name="s",
                                  num_cores=2, num_subcores=NT)


def _make_deg_kernel(n_pad, n_acc, n_chunks):
    rows_per_tile = n_acc // NT
    out_rows_per_tile = n_pad // NT

    @functools.partial(
        pl.kernel,
        out_type=jax.ShapeDtypeStruct((2 * n_pad, HALF), jnp.float32),
        mesh=_sc_mesh(),
        scratch_types=[
            pltpu.VMEM((n_chunks, CHUNK), jnp.int32),
            pltpu.VMEM((CHUNK, HALF), jnp.float32),
            pltpu.VMEM_SHARED((n_acc, HALF), jnp.float32),
            pltpu.SemaphoreType.DMA,
        ],
    )
    def deg_kernel(idx_hbm, ones_hbm, z_hbm, out_hbm, idx_v, ones_v, acc,
                   sem):
        c = lax.axis_index("c")
        s = lax.axis_index("s")

        pltpu.sync_copy(z_hbm, acc.at[pl.ds(s * rows_per_tile,
                                            rows_per_tile)])
        pltpu.sync_copy(ones_hbm, ones_v)
        pltpu.sync_copy(idx_hbm.at[c * NT + s], idx_v)

        plsc.subcore_barrier()

        def body(j, _):
            pltpu.async_copy(ones_v, acc.at[idx_v.at[j]], sem, add=True)
            return _
        lax.fori_loop(0, n_chunks, body, None)

        def drain(j, _):
            pltpu.make_async_copy(ones_v, acc.at[idx_v.at[j]], sem).wait()
            return _
        lax.fori_loop(0, n_chunks, drain, None)

        plsc.subcore_barrier()

        pltpu.sync_copy(
            acc.at[pl.ds(s * out_rows_per_tile, out_rows_per_tile)],
            out_hbm.at[pl.ds(c * n_pad + s * out_rows_per_tile,
                             out_rows_per_tile)])

    return deg_kernel


def _make_agg_kernel(n_pad, n_acc, n_chunks):
    rows_per_tile = n_acc // NT
    out_rows_per_tile = n_pad // NT
    blocks = n_chunks // IBLK

    @functools.partial(
        pl.kernel,
        out_type=jax.ShapeDtypeStruct((2 * n_pad, HALF), jnp.float32),
        mesh=_sc_mesh(),
        scratch_types=[
            pltpu.VMEM((IBLK * CHUNK,), jnp.int32),
            pltpu.VMEM((IBLK, CHUNK), jnp.int32),
            pltpu.VMEM((CHUNK, HALF), jnp.float32),
            pltpu.VMEM((CHUNK, HALF), jnp.float32),
            pltpu.VMEM_SHARED((n_acc, HALF), jnp.float32),
            pltpu.SemaphoreType.DMA,
            pltpu.SemaphoreType.DMA,
        ],
    )
    def agg_kernel(h_hbm, src_hbm, dst_hbm, z_hbm, out_hbm,
                   src_v, dst_v, buf0, buf1, acc, sem0, sem1):
        c = lax.axis_index("c")
        s = lax.axis_index("s")

        pltpu.sync_copy(z_hbm, acc.at[pl.ds(s * rows_per_tile,
                                            rows_per_tile)])
        plsc.subcore_barrier()

        sub = CHUNK // SPLIT

        def gstart(g, buf, sem):
            for p in range(SPLIT):
                pltpu.async_copy(
                    h_hbm.at[src_v.at[pl.ds(g * CHUNK + p * sub, sub)]],
                    buf.at[pl.ds(p * sub, sub)], sem)

        def gwait(g, buf, sem):
            for p in range(SPLIT):
                pltpu.make_async_copy(
                    h_hbm.at[src_v.at[pl.ds(g * CHUNK + p * sub, sub)]],
                    buf.at[pl.ds(p * sub, sub)], sem).wait()

        def block(b, _):
            off = (c * NT + s) * n_chunks + b * IBLK
            pltpu.sync_copy(src_hbm.at[pl.ds(off * CHUNK, IBLK * CHUNK)],
                            src_v)
            pltpu.sync_copy(
                dst_hbm.at[pl.ds(s * n_chunks + b * IBLK, IBLK)], dst_v)
            gstart(0, buf0, sem0)

            def body(i, _):
                g = 2 * i
                gstart(g + 1, buf1, sem1)
                gwait(g, buf0, sem0)
                pltpu.sync_copy(buf0, acc.at[dst_v.at[g]], add=True)

                @pl.when(g + 2 < IBLK)
                def _():
                    gstart(g + 2, buf0, sem0)

                gwait(g + 1, buf1, sem1)
                pltpu.sync_copy(buf1, acc.at[dst_v.at[g + 1]], add=True)
                return _

            lax.fori_loop(0, IBLK // 2, body, None)
            return _

        lax.fori_loop(0, blocks, block, None)
        plsc.subcore_barrier()
        pltpu.sync_copy(
            acc.at[pl.ds(s * out_rows_per_tile, out_rows_per_tile)],
            out_hbm.at[pl.ds(c * n_pad + s * out_rows_per_tile,
                             out_rows_per_tile)])

    return agg_kernel


def _norm(deg):
    return lax.rsqrt(jnp.clip(deg, 1.0, None))


def _mm_body(x_ref, w_ref, o_ref):
    hw = jnp.dot(x_ref[...], w_ref[...], preferred_element_type=jnp.float32)
    o_ref[0] = hw[:, :HALF]
    o_ref[1] = hw[:, HALF:]


def _scale_body(deg_s_ref, m_ref, o_ref):
    ns = _norm(deg_s_ref[...])
    o_ref[...] = m_ref[...] * ns[None, :, None]


def _mid_body(deg_d_ref, deg_s_ref, m_ref, b_ref, w_ref, o_ref):
    nd = _norm(deg_d_ref[...])
    m = jnp.concatenate([m_ref[0], m_ref[1]], axis=1)
    x = jax.nn.relu(m * nd[:, None] + b_ref[...][None, :])
    ns = _norm(deg_s_ref[...])
    hw = jnp.dot(x * ns[:, None], w_ref[...],
                 preferred_element_type=jnp.float32)
    o_ref[0] = hw[:, :HALF]
    o_ref[1] = hw[:, HALF:]


def _post_body(deg_d_ref, m_ref, b_ref, o_ref):
    nd = _norm(deg_d_ref[...])
    m = jnp.concatenate([m_ref[0], m_ref[1]], axis=1)
    o_ref[...] = m * nd[:, None] + b_ref[...][None, :]


def _tc_mm(x, w, n_pad, blk):
    d = x.shape[1]
    return pl.pallas_call(
        _mm_body,
        grid=(n_pad // blk,),
        in_specs=[
            pl.BlockSpec((blk, d), lambda i: (i, 0)),
            pl.BlockSpec((d, d), lambda i: (0, 0)),
        ],
        out_specs=pl.BlockSpec((2, blk, HALF), lambda i: (0, i, 0)),
        out_shape=jax.ShapeDtypeStruct((2, n_pad, HALF), jnp.float32),
    )(x, w)


def _tc_scale(deg_s, m, n_pad, blk):
    return pl.pallas_call(
        _scale_body,
        grid=(n_pad // blk,),
        in_specs=[
            pl.BlockSpec((blk,), lambda i: (i,)),
            pl.BlockSpec((2, blk, HALF), lambda i: (0, i, 0)),
        ],
        out_specs=pl.BlockSpec((2, blk, HALF), lambda i: (0, i, 0)),
        out_shape=jax.ShapeDtypeStruct((2, n_pad, HALF), jnp.float32),
    )(deg_s, m)


def _tc_mid(deg_d, deg_s, m, b, w, n_pad, blk):
    d = w.shape[0]
    return pl.pallas_call(
        _mid_body,
        grid=(n_pad // blk,),
        in_specs=[
            pl.BlockSpec((blk,), lambda i: (i,)),
            pl.BlockSpec((blk,), lambda i: (i,)),
            pl.BlockSpec((2, blk, HALF), lambda i: (0, i, 0)),
            pl.BlockSpec((d,), lambda i: (0,)),
            pl.BlockSpec((d, d), lambda i: (0, 0)),
        ],
        out_specs=pl.BlockSpec((2, blk, HALF), lambda i: (0, i, 0)),
        out_shape=jax.ShapeDtypeStruct((2, n_pad, HALF), jnp.float32),
    )(deg_d, deg_s, m, b, w)


def _tc_post(deg_d, m, b, n_pad, blk):
    d = b.shape[0]
    return pl.pallas_call(
        _post_body,
        grid=(n_pad // blk,),
        in_specs=[
            pl.BlockSpec((blk,), lambda i: (i,)),
            pl.BlockSpec((2, blk, HALF), lambda i: (0, i, 0)),
            pl.BlockSpec((d,), lambda i: (0,)),
        ],
        out_specs=pl.BlockSpec((blk, d), lambda i: (i, 0)),
        out_shape=jax.ShapeDtypeStruct((n_pad, d), jnp.float32),
    )(deg_d, m, b)


def kernel(features, edge_index, W1, b1, W2, b2, W3, b3):
    n, d = features.shape
    e = edge_index.shape[1]
    assert d == 2 * HALF and e % NT == 0

    blk = 512
    n_pad = ((n + blk - 1) // blk) * blk
    n_acc = n_pad + 128
    dummy = n_pad
    e_tile = e // NT
    n_chunks = -(-e_tile // CHUNK)
    n_chunks = -(-n_chunks // IBLK) * IBLK

    src = edge_index[0]
    dst = edge_index[1]

    def pad_tiles(idx, fill):
        idx = idx.reshape(NT, e_tile)
        pad = jnp.full((NT, n_chunks * CHUNK - e_tile), fill, jnp.int32)
        return jnp.concatenate([idx, pad], axis=1).reshape(NT, n_chunks, CHUNK)

    src_gather = pad_tiles(src, 0)
    src_count = pad_tiles(src, dummy)
    dst_pad = pad_tiles(dst, dummy)

    src_agg = jnp.concatenate(
        [src_gather, src_gather + n_pad]).reshape(2 * NT * n_chunks * CHUNK)
    dst_flat = dst_pad.reshape(NT * n_chunks, CHUNK)
    idx_deg = jnp.concatenate([src_count, dst_pad])

    ones_in = jnp.ones((CHUNK, HALF), jnp.float32)
    z_in = jnp.zeros((n_acc // NT, HALF), jnp.float32)

    deg_kernel = _make_deg_kernel(n_pad, n_acc, n_chunks)
    agg_kernel = _make_agg_kernel(n_pad, n_acc, n_chunks)

    x = jnp.pad(features, ((0, n_pad - n), (0, 0)))

    deg = deg_kernel(idx_deg, ones_in, z_in)
    deg_s = deg[:n_pad, 0]
    deg_d = deg[n_pad:, 0]

    def agg(h):
        m = agg_kernel(h.reshape(2 * n_pad, HALF), src_agg, dst_flat, z_in)
        return m.reshape(2, n_pad, HALF)

    h = _tc_scale(deg_s, _tc_mm(x, W1, n_pad, blk), n_pad, blk)
    m = agg(h)
    h = _tc_mid(deg_d, deg_s, m, b1, W2, n_pad, blk)
    m = agg(h)
    h = _tc_mid(deg_d, deg_s, m, b2, W3, n_pad, blk)
    m = agg(h)
    out = _tc_post(deg_d, m, b3, n_pad, blk)

    return out[:n]


# speedup vs baseline: 1.0214x; 1.0214x over previous
import functools

import jax
import jax.numpy as jnp
from jax import lax
from jax.experimental import pallas as pl
from jax.experimental.pallas import tpu as pltpu
from jax.experimental.pallas import tpu_sc as plsc

NT = 16
CHUNK = 128
HALF = 128
IBLK = 40
SPLIT = 1


def _sc_mesh():
    return plsc.VectorSubcoreMesh(core_axis_name="c", subcore_axis_name="s",
                                  num_cores=2, num_subcores=NT)


def _make_deg_kernel(n_pad, n_acc, n_chunks):
    rows_per_tile = n_acc // NT
    out_rows_per_tile = n_pad // NT

    @functools.partial(
        pl.kernel,
        out_type=jax.ShapeDtypeStruct((2 * n_pad, HALF), jnp.float32),
        mesh=_sc_mesh(),
        scratch_types=[
            pltpu.VMEM((n_chunks, CHUNK), jnp.int32),
            pltpu.VMEM((CHUNK, HALF), jnp.float32),
            pltpu.VMEM_SHARED((n_acc, HALF), jnp.float32),
            pltpu.SemaphoreType.DMA,
        ],
    )
    def deg_kernel(idx_hbm, ones_hbm, z_hbm, out_hbm, idx_v, ones_v, acc,
                   sem):
        c = lax.axis_index("c")
        s = lax.axis_index("s")

        pltpu.sync_copy(z_hbm, acc.at[pl.ds(s * rows_per_tile,
                                            rows_per_tile)])
        pltpu.sync_copy(ones_hbm, ones_v)
        pltpu.sync_copy(idx_hbm.at[c * NT + s], idx_v)

        plsc.subcore_barrier()

        def body(j, _):
            pltpu.async_copy(ones_v, acc.at[idx_v.at[j]], sem, add=True)
            return _
        lax.fori_loop(0, n_chunks, body, None)

        def drain(j, _):
            pltpu.make_async_copy(ones_v, acc.at[idx_v.at[j]], sem).wait()
            return _
        lax.fori_loop(0, n_chunks, drain, None)

        plsc.subcore_barrier()

        pltpu.sync_copy(
            acc.at[pl.ds(s * out_rows_per_tile, out_rows_per_tile)],
            out_hbm.at[pl.ds(c * n_pad + s * out_rows_per_tile,
                             out_rows_per_tile)])

    return deg_kernel


def _make_agg_kernel(n_pad, n_acc, n_chunks):
    rows_per_tile = n_acc // NT
    out_rows_per_tile = n_pad // NT
    blocks = n_chunks // IBLK

    @functools.partial(
        pl.kernel,
        out_type=jax.ShapeDtypeStruct((2 * n_pad, HALF), jnp.float32),
        mesh=_sc_mesh(),
        scratch_types=[
            pltpu.VMEM((IBLK * CHUNK,), jnp.int32),
            pltpu.VMEM((IBLK, CHUNK), jnp.int32),
            pltpu.VMEM((CHUNK, HALF), jnp.float32),
            pltpu.VMEM((CHUNK, HALF), jnp.float32),
            pltpu.VMEM_SHARED((n_acc, HALF), jnp.float32),
            pltpu.SemaphoreType.DMA,
            pltpu.SemaphoreType.DMA,
        ],
    )
    def agg_kernel(h_hbm, src_hbm, dst_hbm, z_hbm, out_hbm,
                   src_v, dst_v, buf0, buf1, acc, sem0, sem1):
        c = lax.axis_index("c")
        s = lax.axis_index("s")

        pltpu.sync_copy(z_hbm, acc.at[pl.ds(s * rows_per_tile,
                                            rows_per_tile)])
        plsc.subcore_barrier()

        sub = CHUNK // SPLIT

        def gstart(g, buf, sem):
            for p in range(SPLIT):
                pltpu.async_copy(
                    h_hbm.at[src_v.at[pl.ds(g * CHUNK + p * sub, sub)]],
                    buf.at[pl.ds(p * sub, sub)], sem)

        def gwait(g, buf, sem):
            for p in range(SPLIT):
                pltpu.make_async_copy(
                    h_hbm.at[src_v.at[pl.ds(g * CHUNK + p * sub, sub)]],
                    buf.at[pl.ds(p * sub, sub)], sem).wait()

        def block(b, _):
            off = (c * NT + s) * n_chunks + b * IBLK
            pltpu.sync_copy(src_hbm.at[pl.ds(off * CHUNK, IBLK * CHUNK)],
                            src_v)
            pltpu.sync_copy(
                dst_hbm.at[pl.ds(s * n_chunks + b * IBLK, IBLK)], dst_v)
            gstart(0, buf0, sem0)

            def body(i, _):
                g = 2 * i
                gstart(g + 1, buf1, sem1)
                gwait(g, buf0, sem0)
                pltpu.sync_copy(buf0, acc.at[dst_v.at[g]], add=True)

                @pl.when(g + 2 < IBLK)
                def _():
                    gstart(g + 2, buf0, sem0)

                gwait(g + 1, buf1, sem1)
                pltpu.sync_copy(buf1, acc.at[dst_v.at[g + 1]], add=True)
                return _

            lax.fori_loop(0, IBLK // 2, body, None)
            return _

        lax.fori_loop(0, blocks, block, None)
        plsc.subcore_barrier()
        pltpu.sync_copy(
            acc.at[pl.ds(s * out_rows_per_tile, out_rows_per_tile)],
            out_hbm.at[pl.ds(c * n_pad + s * out_rows_per_tile,
                             out_rows_per_tile)])

    return agg_kernel


def _norm(deg):
    return lax.rsqrt(jnp.clip(deg, 1.0, None))


def _mm_body(x_ref, w_ref, o_ref):
    hw = jnp.dot(x_ref[...], w_ref[...], preferred_element_type=jnp.float32)
    o_ref[0] = hw[:, :HALF]
    o_ref[1] = hw[:, HALF:]


def _scale_body(deg_s_ref, m_ref, o_ref):
    ns = _norm(deg_s_ref[...])
    o_ref[...] = m_ref[...] * ns[None, :, None]


def _mid_body(deg_d_ref, deg_s_ref, m_ref, b_ref, w_ref, o_ref):
    nd = _norm(deg_d_ref[...])
    m = jnp.concatenate([m_ref[0], m_ref[1]], axis=1)
    x = jax.nn.relu(m * nd[:, None] + b_ref[...][None, :])
    ns = _norm(deg_s_ref[...])
    hw = jnp.dot(x * ns[:, None], w_ref[...],
                 preferred_element_type=jnp.float32)
    o_ref[0] = hw[:, :HALF]
    o_ref[1] = hw[:, HALF:]


def _post_body(deg_d_ref, m_ref, b_ref, o_ref):
    nd = _norm(deg_d_ref[...])
    m = jnp.concatenate([m_ref[0], m_ref[1]], axis=1)
    o_ref[...] = m * nd[:, None] + b_ref[...][None, :]


def _tc_mm(x, w, n_pad, blk):
    d = x.shape[1]
    return pl.pallas_call(
        _mm_body,
        grid=(n_pad // blk,),
        in_specs=[
            pl.BlockSpec((blk, d), lambda i: (i, 0)),
            pl.BlockSpec((d, d), lambda i: (0, 0)),
        ],
        out_specs=pl.BlockSpec((2, blk, HALF), lambda i: (0, i, 0)),
        out_shape=jax.ShapeDtypeStruct((2, n_pad, HALF), jnp.float32),
    )(x, w)


def _tc_scale(deg_s, m, n_pad, blk):
    return pl.pallas_call(
        _scale_body,
        grid=(n_pad // blk,),
        in_specs=[
            pl.BlockSpec((blk,), lambda i: (i,)),
            pl.BlockSpec((2, blk, HALF), lambda i: (0, i, 0)),
        ],
        out_specs=pl.BlockSpec((2, blk, HALF), lambda i: (0, i, 0)),
        out_shape=jax.ShapeDtypeStruct((2, n_pad, HALF), jnp.float32),
    )(deg_s, m)


def _tc_mid(deg_d, deg_s, m, b, w, n_pad, blk):
    d = w.shape[0]
    return pl.pallas_call(
        _mid_body,
        grid=(n_pad // blk,),
        in_specs=[
            pl.BlockSpec((blk,), lambda i: (i,)),
            pl.BlockSpec((blk,), lambda i: (i,)),
            pl.BlockSpec((2, blk, HALF), lambda i: (0, i, 0)),
            pl.BlockSpec((d,), lambda i: (0,)),
            pl.BlockSpec((d, d), lambda i: (0, 0)),
        ],
        out_specs=pl.BlockSpec((2, blk, HALF), lambda i: (0, i, 0)),
        out_shape=jax.ShapeDtypeStruct((2, n_pad, HALF), jnp.float32),
    )(deg_d, deg_s, m, b, w)


def _tc_post(deg_d, m, b, n_pad, blk):
    d = b.shape[0]
    return pl.pallas_call(
        _post_body,
        grid=(n_pad // blk,),
        in_specs=[
            pl.BlockSpec((blk,), lambda i: (i,)),
            pl.BlockSpec((2, blk, HALF), lambda i: (0, i, 0)),
            pl.BlockSpec((d,), lambda i: (0,)),
        ],
        out_specs=pl.BlockSpec((blk, d), lambda i: (i, 0)),
        out_shape=jax.ShapeDtypeStruct((n_pad, d), jnp.float32),
    )(deg_d, m, b)


def kernel(features, edge_index, W1, b1, W2, b2, W3, b3):
    n, d = features.shape
    e = edge_index.shape[1]
    assert d == 2 * HALF and e % NT == 0

    blk = 1024
    n_pad = ((n + blk - 1) // blk) * blk
    n_acc = n_pad + 128
    dummy = n_pad
    e_tile = e // NT
    n_chunks = -(-e_tile // CHUNK)
    n_chunks = -(-n_chunks // IBLK) * IBLK

    src = edge_index[0]
    dst = edge_index[1]

    def pad_tiles(idx, fill):
        idx = idx.reshape(NT, e_tile)
        pad = jnp.full((NT, n_chunks * CHUNK - e_tile), fill, jnp.int32)
        return jnp.concatenate([idx, pad], axis=1).reshape(NT, n_chunks, CHUNK)

    src_gather = pad_tiles(src, 0)
    src_count = pad_tiles(src, dummy)
    dst_pad = pad_tiles(dst, dummy)

    src_agg = jnp.concatenate(
        [src_gather, src_gather + n_pad]).reshape(2 * NT * n_chunks * CHUNK)
    dst_flat = dst_pad.reshape(NT * n_chunks, CHUNK)
    idx_deg = jnp.concatenate([src_count, dst_pad])

    ones_in = jnp.ones((CHUNK, HALF), jnp.float32)
    z_in = jnp.zeros((n_acc // NT, HALF), jnp.float32)

    deg_kernel = _make_deg_kernel(n_pad, n_acc, n_chunks)
    agg_kernel = _make_agg_kernel(n_pad, n_acc, n_chunks)

    x = jnp.pad(features, ((0, n_pad - n), (0, 0)))

    deg = deg_kernel(idx_deg, ones_in, z_in)
    deg_s = deg[:n_pad, 0]
    deg_d = deg[n_pad:, 0]

    def agg(h):
        m = agg_kernel(h.reshape(2 * n_pad, HALF), src_agg, dst_flat, z_in)
        return m.reshape(2, n_pad, HALF)

    h = _tc_scale(deg_s, _tc_mm(x, W1, n_pad, blk), n_pad, blk)
    m = agg(h)
    h = _tc_mid(deg_d, deg_s, m, b1, W2, n_pad, blk)
    m = agg(h)
    h = _tc_mid(deg_d, deg_s, m, b2, W3, n_pad, blk)
    m = agg(h)
    out = _tc_post(deg_d, m, b3, n_pad, blk)

    return out[:n]


# speedup vs baseline: 1.0307x; 1.0091x over previous
import functools

import jax
import jax.numpy as jnp
from jax import lax
from jax.experimental import pallas as pl
from jax.experimental.pallas import tpu as pltpu
from jax.experimental.pallas import tpu_sc as plsc

NT = 16
CHUNK = 128
HALF = 128
IBLK = 40
SPLIT = 1


def _sc_mesh():
    return plsc.VectorSubcoreMesh(core_axis_name="c", subcore_axis_name="s",
                                  num_cores=2, num_subcores=NT)


def _make_deg_kernel(n_pad, n_acc, n_chunks):
    rows_per_tile = n_acc // NT
    out_rows_per_tile = n_pad // NT

    @functools.partial(
        pl.kernel,
        out_type=jax.ShapeDtypeStruct((2 * n_pad, HALF), jnp.float32),
        mesh=_sc_mesh(),
        scratch_types=[
            pltpu.VMEM((n_chunks, CHUNK), jnp.int32),
            pltpu.VMEM((CHUNK, HALF), jnp.float32),
            pltpu.VMEM_SHARED((n_acc, HALF), jnp.float32),
            pltpu.SemaphoreType.DMA,
        ],
    )
    def deg_kernel(idx_hbm, ones_hbm, z_hbm, out_hbm, idx_v, ones_v, acc,
                   sem):
        c = lax.axis_index("c")
        s = lax.axis_index("s")

        pltpu.sync_copy(z_hbm, acc.at[pl.ds(s * rows_per_tile,
                                            rows_per_tile)])
        pltpu.sync_copy(ones_hbm, ones_v)
        pltpu.sync_copy(idx_hbm.at[c * NT + s], idx_v)

        plsc.subcore_barrier()

        def body(j, _):
            pltpu.async_copy(ones_v, acc.at[idx_v.at[j]], sem, add=True)
            return _
        lax.fori_loop(0, n_chunks, body, None)

        def drain(j, _):
            pltpu.make_async_copy(ones_v, acc.at[idx_v.at[j]], sem).wait()
            return _
        lax.fori_loop(0, n_chunks, drain, None)

        plsc.subcore_barrier()

        pltpu.sync_copy(
            acc.at[pl.ds(s * out_rows_per_tile, out_rows_per_tile)],
            out_hbm.at[pl.ds(c * n_pad + s * out_rows_per_tile,
                             out_rows_per_tile)])

    return deg_kernel


def _make_agg_kernel(n_pad, n_acc, n_chunks):
    rows_per_tile = n_acc // NT
    out_rows_per_tile = n_pad // NT
    blocks = n_chunks // IBLK

    @functools.partial(
        pl.kernel,
        out_type=jax.ShapeDtypeStruct((2 * n_pad, HALF), jnp.float32),
        mesh=_sc_mesh(),
        scratch_types=[
            pltpu.VMEM((IBLK * CHUNK,), jnp.int32),
            pltpu.VMEM((IBLK, CHUNK), jnp.int32),
            pltpu.VMEM((CHUNK, HALF), jnp.float32),
            pltpu.VMEM((CHUNK, HALF), jnp.float32),
            pltpu.VMEM_SHARED((n_acc, HALF), jnp.float32),
            pltpu.SemaphoreType.DMA,
            pltpu.SemaphoreType.DMA,
        ],
    )
    def agg_kernel(h_hbm, src_hbm, dst_hbm, z_hbm, out_hbm,
                   src_v, dst_v, buf0, buf1, acc, sem0, sem1):
        c = lax.axis_index("c")
        s = lax.axis_index("s")

        pltpu.sync_copy(z_hbm, acc.at[pl.ds(s * rows_per_tile,
                                            rows_per_tile)])
        plsc.subcore_barrier()

        sub = CHUNK // SPLIT

        def gstart(g, buf, sem):
            for p in range(SPLIT):
                pltpu.async_copy(
                    h_hbm.at[src_v.at[pl.ds(g * CHUNK + p * sub, sub)]],
                    buf.at[pl.ds(p * sub, sub)], sem)

        def gwait(g, buf, sem):
            for p in range(SPLIT):
                pltpu.make_async_copy(
                    h_hbm.at[src_v.at[pl.ds(g * CHUNK + p * sub, sub)]],
                    buf.at[pl.ds(p * sub, sub)], sem).wait()

        def block(b, _):
            off = (c * NT + s) * n_chunks + b * IBLK
            pltpu.sync_copy(src_hbm.at[pl.ds(off * CHUNK, IBLK * CHUNK)],
                            src_v)
            pltpu.sync_copy(
                dst_hbm.at[pl.ds(s * n_chunks + b * IBLK, IBLK)], dst_v)
            gstart(0, buf0, sem0)

            def body(i, _):
                g = 2 * i
                gstart(g + 1, buf1, sem1)
                gwait(g, buf0, sem0)
                pltpu.sync_copy(buf0, acc.at[dst_v.at[g]], add=True)

                @pl.when(g + 2 < IBLK)
                def _():
                    gstart(g + 2, buf0, sem0)

                gwait(g + 1, buf1, sem1)
                pltpu.sync_copy(buf1, acc.at[dst_v.at[g + 1]], add=True)
                return _

            lax.fori_loop(0, IBLK // 2, body, None)
            return _

        lax.fori_loop(0, blocks, block, None)
        plsc.subcore_barrier()
        pltpu.sync_copy(
            acc.at[pl.ds(s * out_rows_per_tile, out_rows_per_tile)],
            out_hbm.at[pl.ds(c * n_pad + s * out_rows_per_tile,
                             out_rows_per_tile)])

    return agg_kernel


def _norm(deg):
    return lax.rsqrt(jnp.clip(deg, 1.0, None))


def _mm_body(x_ref, w_ref, o_ref):
    hw = jnp.dot(x_ref[...], w_ref[...], preferred_element_type=jnp.float32)
    o_ref[0] = hw[:, :HALF]
    o_ref[1] = hw[:, HALF:]


def _scale_body(deg_s_ref, m_ref, o_ref):
    ns = _norm(deg_s_ref[...])
    o_ref[...] = m_ref[...] * ns[None, :, None]


def _mid_body(deg_d_ref, deg_s_ref, m_ref, b_ref, w_ref, o_ref):
    nd = _norm(deg_d_ref[...])
    m = jnp.concatenate([m_ref[0], m_ref[1]], axis=1)
    x = jax.nn.relu(m * nd[:, None] + b_ref[...][None, :])
    ns = _norm(deg_s_ref[...])
    hw = jnp.dot(x * ns[:, None], w_ref[...],
                 preferred_element_type=jnp.float32)
    o_ref[0] = hw[:, :HALF]
    o_ref[1] = hw[:, HALF:]


def _post_body(deg_d_ref, m_ref, b_ref, o_ref):
    nd = _norm(deg_d_ref[...])
    m = jnp.concatenate([m_ref[0], m_ref[1]], axis=1)
    o_ref[...] = m * nd[:, None] + b_ref[...][None, :]


def _tc_mm(x, w, n_pad, blk):
    d = x.shape[1]
    return pl.pallas_call(
        _mm_body,
        grid=(n_pad // blk,),
        in_specs=[
            pl.BlockSpec((blk, d), lambda i: (i, 0)),
            pl.BlockSpec((d, d), lambda i: (0, 0)),
        ],
        out_specs=pl.BlockSpec((2, blk, HALF), lambda i: (0, i, 0)),
        out_shape=jax.ShapeDtypeStruct((2, n_pad, HALF), jnp.float32),
    )(x, w)


def _tc_scale(deg_s, m, n_pad, blk):
    return pl.pallas_call(
        _scale_body,
        grid=(n_pad // blk,),
        in_specs=[
            pl.BlockSpec((blk,), lambda i: (i,)),
            pl.BlockSpec((2, blk, HALF), lambda i: (0, i, 0)),
        ],
        out_specs=pl.BlockSpec((2, blk, HALF), lambda i: (0, i, 0)),
        out_shape=jax.ShapeDtypeStruct((2, n_pad, HALF), jnp.float32),
    )(deg_s, m)


def _tc_mid(deg_d, deg_s, m, b, w, n_pad, blk):
    d = w.shape[0]
    return pl.pallas_call(
        _mid_body,
        grid=(n_pad // blk,),
        in_specs=[
            pl.BlockSpec((blk,), lambda i: (i,)),
            pl.BlockSpec((blk,), lambda i: (i,)),
            pl.BlockSpec((2, blk, HALF), lambda i: (0, i, 0)),
            pl.BlockSpec((d,), lambda i: (0,)),
            pl.BlockSpec((d, d), lambda i: (0, 0)),
        ],
        out_specs=pl.BlockSpec((2, blk, HALF), lambda i: (0, i, 0)),
        out_shape=jax.ShapeDtypeStruct((2, n_pad, HALF), jnp.float32),
    )(deg_d, deg_s, m, b, w)


def _tc_post(deg_d, m, b, n_pad, blk):
    d = b.shape[0]
    return pl.pallas_call(
        _post_body,
        grid=(n_pad // blk,),
        in_specs=[
            pl.BlockSpec((blk,), lambda i: (i,)),
            pl.BlockSpec((2, blk, HALF), lambda i: (0, i, 0)),
            pl.BlockSpec((d,), lambda i: (0,)),
        ],
        out_specs=pl.BlockSpec((blk, d), lambda i: (i, 0)),
        out_shape=jax.ShapeDtypeStruct((n_pad, d), jnp.float32),
    )(deg_d, m, b)


def kernel(features, edge_index, W1, b1, W2, b2, W3, b3):
    n, d = features.shape
    e = edge_index.shape[1]
    assert d == 2 * HALF and e % NT == 0

    blk = 2048
    n_pad = ((n + blk - 1) // blk) * blk
    n_acc = n_pad + 128
    dummy = n_pad
    e_tile = e // NT
    n_chunks = -(-e_tile // CHUNK)
    n_chunks = -(-n_chunks // IBLK) * IBLK

    src = edge_index[0]
    dst = edge_index[1]

    def pad_tiles(idx, fill):
        idx = idx.reshape(NT, e_tile)
        pad = jnp.full((NT, n_chunks * CHUNK - e_tile), fill, jnp.int32)
        return jnp.concatenate([idx, pad], axis=1).reshape(NT, n_chunks, CHUNK)

    src_gather = pad_tiles(src, 0)
    src_count = pad_tiles(src, dummy)
    dst_pad = pad_tiles(dst, dummy)

    src_agg = jnp.concatenate(
        [src_gather, src_gather + n_pad]).reshape(2 * NT * n_chunks * CHUNK)
    dst_flat = dst_pad.reshape(NT * n_chunks, CHUNK)
    idx_deg = jnp.concatenate([src_count, dst_pad])

    ones_in = jnp.ones((CHUNK, HALF), jnp.float32)
    z_in = jnp.zeros((n_acc // NT, HALF), jnp.float32)

    deg_kernel = _make_deg_kernel(n_pad, n_acc, n_chunks)
    agg_kernel = _make_agg_kernel(n_pad, n_acc, n_chunks)

    x = jnp.pad(features, ((0, n_pad - n), (0, 0)))

    deg = deg_kernel(idx_deg, ones_in, z_in)
    deg_s = deg[:n_pad, 0]
    deg_d = deg[n_pad:, 0]

    def agg(h):
        m = agg_kernel(h.reshape(2 * n_pad, HALF), src_agg, dst_flat, z_in)
        return m.reshape(2, n_pad, HALF)

    h = _tc_scale(deg_s, _tc_mm(x, W1, n_pad, blk), n_pad, blk)
    m = agg(h)
    h = _tc_mid(deg_d, deg_s, m, b1, W2, n_pad, blk)
    m = agg(h)
    h = _tc_mid(deg_d, deg_s, m, b2, W3, n_pad, blk)
    m = agg(h)
    out = _tc_post(deg_d, m, b3, n_pad, blk)

    return out[:n]


# speedup vs baseline: 1.0308x; 1.0001x over previous
import functools

import jax
import jax.numpy as jnp
from jax import lax
from jax.experimental import pallas as pl
from jax.experimental.pallas import tpu as pltpu
from jax.experimental.pallas import tpu_sc as plsc

NT = 16
CHUNK = 128
HALF = 128
IBLK = 40
SPLIT = 1


def _sc_mesh():
    return plsc.VectorSubcoreMesh(core_axis_name="c", subcore_axis_name="s",
                                  num_cores=2, num_subcores=NT)


def _make_deg_kernel(n_pad, n_acc, n_chunks):
    rows_per_tile = n_acc // NT
    out_rows_per_tile = n_pad // NT

    @functools.partial(
        pl.kernel,
        out_type=jax.ShapeDtypeStruct((2 * n_pad, HALF), jnp.float32),
        mesh=_sc_mesh(),
        scratch_types=[
            pltpu.VMEM((n_chunks, CHUNK), jnp.int32),
            pltpu.VMEM((CHUNK, HALF), jnp.float32),
            pltpu.VMEM_SHARED((n_acc, HALF), jnp.float32),
            pltpu.SemaphoreType.DMA,
        ],
    )
    def deg_kernel(idx_hbm, ones_hbm, z_hbm, out_hbm, idx_v, ones_v, acc,
                   sem):
        c = lax.axis_index("c")
        s = lax.axis_index("s")

        pltpu.sync_copy(z_hbm, acc.at[pl.ds(s * rows_per_tile,
                                            rows_per_tile)])
        pltpu.sync_copy(ones_hbm, ones_v)
        pltpu.sync_copy(idx_hbm.at[c * NT + s], idx_v)

        plsc.subcore_barrier()

        def body(j, _):
            pltpu.async_copy(ones_v, acc.at[idx_v.at[j]], sem, add=True)
            return _
        lax.fori_loop(0, n_chunks, body, None)

        def drain(j, _):
            pltpu.make_async_copy(ones_v, acc.at[idx_v.at[j]], sem).wait()
            return _
        lax.fori_loop(0, n_chunks, drain, None)

        plsc.subcore_barrier()

        pltpu.sync_copy(
            acc.at[pl.ds(s * out_rows_per_tile, out_rows_per_tile)],
            out_hbm.at[pl.ds(c * n_pad + s * out_rows_per_tile,
                             out_rows_per_tile)])

    return deg_kernel


def _make_agg_kernel(n_pad, n_acc, n_chunks):
    rows_per_tile = n_acc // NT
    out_rows_per_tile = n_pad // NT
    blocks = n_chunks // IBLK

    @functools.partial(
        pl.kernel,
        out_type=jax.ShapeDtypeStruct((2 * n_pad, HALF), jnp.float32),
        mesh=_sc_mesh(),
        scratch_types=[
            pltpu.VMEM((IBLK * CHUNK,), jnp.int32),
            pltpu.VMEM((IBLK, CHUNK), jnp.int32),
            pltpu.VMEM((CHUNK, HALF), jnp.float32),
            pltpu.VMEM((CHUNK, HALF), jnp.float32),
            pltpu.VMEM_SHARED((n_acc, HALF), jnp.float32),
            pltpu.SemaphoreType.DMA,
            pltpu.SemaphoreType.DMA,
        ],
    )
    def agg_kernel(h_hbm, src_hbm, dst_hbm, z_hbm, out_hbm,
                   src_v, dst_v, buf0, buf1, acc, sem0, sem1):
        c = lax.axis_index("c")
        s = lax.axis_index("s")

        pltpu.sync_copy(z_hbm, acc.at[pl.ds(s * rows_per_tile,
                                            rows_per_tile)])
        plsc.subcore_barrier()

        sub = CHUNK // SPLIT

        def gstart(g, buf, sem):
            for p in range(SPLIT):
                pltpu.async_copy(
                    h_hbm.at[src_v.at[pl.ds(g * CHUNK + p * sub, sub)]],
                    buf.at[pl.ds(p * sub, sub)], sem)

        def gwait(g, buf, sem):
            for p in range(SPLIT):
                pltpu.make_async_copy(
                    h_hbm.at[src_v.at[pl.ds(g * CHUNK + p * sub, sub)]],
                    buf.at[pl.ds(p * sub, sub)], sem).wait()

        def block(b, _):
            off = (c * NT + s) * n_chunks + b * IBLK
            pltpu.sync_copy(src_hbm.at[pl.ds(off * CHUNK, IBLK * CHUNK)],
                            src_v)
            pltpu.sync_copy(
                dst_hbm.at[pl.ds(s * n_chunks + b * IBLK, IBLK)], dst_v)
            gstart(0, buf0, sem0)

            def body(i, _):
                g = 2 * i
                gstart(g + 1, buf1, sem1)
                gwait(g, buf0, sem0)
                pltpu.sync_copy(buf0, acc.at[dst_v.at[g]], add=True)

                @pl.when(g + 2 < IBLK)
                def _():
                    gstart(g + 2, buf0, sem0)

                gwait(g + 1, buf1, sem1)
                pltpu.sync_copy(buf1, acc.at[dst_v.at[g + 1]], add=True)
                return _

            lax.fori_loop(0, IBLK // 2, body, None)
            return _

        lax.fori_loop(0, blocks, block, None)
        plsc.subcore_barrier()
        pltpu.sync_copy(
            acc.at[pl.ds(s * out_rows_per_tile, out_rows_per_tile)],
            out_hbm.at[pl.ds(c * n_pad + s * out_rows_per_tile,
                             out_rows_per_tile)])

    return agg_kernel


def _norm(deg):
    return lax.rsqrt(jnp.clip(deg, 1.0, None))


def _mm_body(x_ref, w_ref, o_ref):
    hw = jnp.dot(x_ref[...], w_ref[...], preferred_element_type=jnp.float32)
    o_ref[0] = hw[:, :HALF]
    o_ref[1] = hw[:, HALF:]


def _scale_body(deg_s_ref, m_ref, o_ref):
    ns = _norm(deg_s_ref[...])
    o_ref[...] = m_ref[...] * ns[None, :, None]


def _mid_body(deg_d_ref, deg_s_ref, m_ref, b_ref, w_ref, o_ref):
    nd = _norm(deg_d_ref[...])
    m = jnp.concatenate([m_ref[0], m_ref[1]], axis=1)
    x = jax.nn.relu(m * nd[:, None] + b_ref[...][None, :])
    ns = _norm(deg_s_ref[...])
    hw = jnp.dot(x * ns[:, None], w_ref[...],
                 preferred_element_type=jnp.float32)
    o_ref[0] = hw[:, :HALF]
    o_ref[1] = hw[:, HALF:]


def _post_body(deg_d_ref, m_ref, b_ref, o_ref):
    nd = _norm(deg_d_ref[...])
    m = jnp.concatenate([m_ref[0], m_ref[1]], axis=1)
    o_ref[...] = m * nd[:, None] + b_ref[...][None, :]


def _tc_mm(x, w, n_pad, blk):
    d = x.shape[1]
    return pl.pallas_call(
        _mm_body,
        grid=(n_pad // blk,),
        in_specs=[
            pl.BlockSpec((blk, d), lambda i: (i, 0)),
            pl.BlockSpec((d, d), lambda i: (0, 0)),
        ],
        out_specs=pl.BlockSpec((2, blk, HALF), lambda i: (0, i, 0)),
        out_shape=jax.ShapeDtypeStruct((2, n_pad, HALF), jnp.float32),
    )(x, w)


def _tc_scale(deg_s, m, n_pad, blk):
    return pl.pallas_call(
        _scale_body,
        grid=(n_pad // blk,),
        in_specs=[
            pl.BlockSpec((blk,), lambda i: (i,)),
            pl.BlockSpec((2, blk, HALF), lambda i: (0, i, 0)),
        ],
        out_specs=pl.BlockSpec((2, blk, HALF), lambda i: (0, i, 0)),
        out_shape=jax.ShapeDtypeStruct((2, n_pad, HALF), jnp.float32),
    )(deg_s, m)


def _tc_mid(deg_d, deg_s, m, b, w, n_pad, blk):
    d = w.shape[0]
    return pl.pallas_call(
        _mid_body,
        grid=(n_pad // blk,),
        in_specs=[
            pl.BlockSpec((blk,), lambda i: (i,)),
            pl.BlockSpec((blk,), lambda i: (i,)),
            pl.BlockSpec((2, blk, HALF), lambda i: (0, i, 0)),
            pl.BlockSpec((d,), lambda i: (0,)),
            pl.BlockSpec((d, d), lambda i: (0, 0)),
        ],
        out_specs=pl.BlockSpec((2, blk, HALF), lambda i: (0, i, 0)),
        out_shape=jax.ShapeDtypeStruct((2, n_pad, HALF), jnp.float32),
    )(deg_d, deg_s, m, b, w)


def _tc_post(deg_d, m, b, n_pad, blk):
    d = b.shape[0]
    return pl.pallas_call(
        _post_body,
        grid=(n_pad // blk,),
        in_specs=[
            pl.BlockSpec((blk,), lambda i: (i,)),
            pl.BlockSpec((2, blk, HALF), lambda i: (0, i, 0)),
            pl.BlockSpec((d,), lambda i: (0,)),
        ],
        out_specs=pl.BlockSpec((blk, d), lambda i: (i, 0)),
        out_shape=jax.ShapeDtypeStruct((n_pad, d), jnp.float32),
    )(deg_d, m, b)


def kernel(features, edge_index, W1, b1, W2, b2, W3, b3):
    n, d = features.shape
    e = edge_index.shape[1]
    assert d == 2 * HALF and e % NT == 0

    blk = 10240
    n_pad = ((n + blk - 1) // blk) * blk
    n_acc = n_pad + 128
    dummy = n_pad
    e_tile = e // NT
    n_chunks = -(-e_tile // CHUNK)
    n_chunks = -(-n_chunks // IBLK) * IBLK

    src = edge_index[0]
    dst = edge_index[1]

    def pad_tiles(idx, fill):
        idx = idx.reshape(NT, e_tile)
        pad = jnp.full((NT, n_chunks * CHUNK - e_tile), fill, jnp.int32)
        return jnp.concatenate([idx, pad], axis=1).reshape(NT, n_chunks, CHUNK)

    src_gather = pad_tiles(src, 0)
    src_count = pad_tiles(src, dummy)
    dst_pad = pad_tiles(dst, dummy)

    src_agg = jnp.concatenate(
        [src_gather, src_gather + n_pad]).reshape(2 * NT * n_chunks * CHUNK)
    dst_flat = dst_pad.reshape(NT * n_chunks, CHUNK)
    idx_deg = jnp.concatenate([src_count, dst_pad])

    ones_in = jnp.ones((CHUNK, HALF), jnp.float32)
    z_in = jnp.zeros((n_acc // NT, HALF), jnp.float32)

    deg_kernel = _make_deg_kernel(n_pad, n_acc, n_chunks)
    agg_kernel = _make_agg_kernel(n_pad, n_acc, n_chunks)

    x = jnp.pad(features, ((0, n_pad - n), (0, 0)))

    deg = deg_kernel(idx_deg, ones_in, z_in)
    deg_s = deg[:n_pad, 0]
    deg_d = deg[n_pad:, 0]

    def agg(h):
        m = agg_kernel(h.reshape(2 * n_pad, HALF), src_agg, dst_flat, z_in)
        return m.reshape(2, n_pad, HALF)

    h = _tc_scale(deg_s, _tc_mm(x, W1, n_pad, blk), n_pad, blk)
    m = agg(h)
    h = _tc_mid(deg_d, deg_s, m, b1, W2, n_pad, blk)
    m = agg(h)
    h = _tc_mid(deg_d, deg_s, m, b2, W3, n_pad, blk)
    m = agg(h)
    out = _tc_post(deg_d, m, b3, n_pad, blk)

    return out[:n]
